# Initial kernel scaffold; baseline (speedup 1.0000x reference)
import jax
import jax.numpy as jnp
from jax.experimental import pallas as pl


def kernel(x, edge_index, batch, W1, a_src1, a_dst1, b1, W2, a_src2, a_dst2, b2, W_ih, W_hh, b_ih, b_hh, W_fc, b_fc):
    raise NotImplementedError("write your pallas kernel here")


# speedup vs baseline: 2.9270x; 2.9270x over previous
import functools

import jax
import jax.numpy as jnp
from jax import lax
from jax.experimental import pallas as pl
from jax.experimental.pallas import tpu as pltpu
from jax.experimental.pallas import tpu_sc as plsc

_NW = 32
_B = 128


def _pre_tail(nsteps, hcur, w_ref, as_ref, ad_ref,
              hw_ref, ss_ref, sd_ref, m_ref, mscr):
    i = pl.program_id(0)
    hw = jnp.dot(hcur, w_ref[...], preferred_element_type=jnp.float32)
    hw_ref[...] = hw
    ss = jnp.dot(hw, as_ref[...], preferred_element_type=jnp.float32)
    sd = jnp.dot(hw, ad_ref[...], preferred_element_type=jnp.float32)
    ss_ref[...] = ss
    sd_ref[...] = sd
    bs = jnp.max(ss)
    bd = jnp.max(sd)

    @pl.when(i == 0)
    def _():
        mscr[0] = bs
        mscr[1] = bd

    @pl.when(i > 0)
    def _():
        mscr[0] = jnp.maximum(mscr[0], bs)
        mscr[1] = jnp.maximum(mscr[1], bd)

    @pl.when(i == nsteps - 1)
    def _():
        m = mscr[0] + mscr[1]
        m = jnp.where(m < 0.0, 0.2 * m, m)
        m_ref[...] = jnp.full((8, 128), m, jnp.float32)


def _pre_body(nsteps, x_ref, w_ref, as_ref, ad_ref,
              hw_ref, ss_ref, sd_ref, m_ref, mscr):
    _pre_tail(nsteps, x_ref[...], w_ref, as_ref, ad_ref,
              hw_ref, ss_ref, sd_ref, m_ref, mscr)


def _dense_outs(n, h, blk):
    return (
        [
            pl.BlockSpec((blk, h), lambda i: (i, 0)),
            pl.BlockSpec((blk, 1), lambda i: (i, 0)),
            pl.BlockSpec((blk, 1), lambda i: (i, 0)),
            pl.BlockSpec((8, 128), lambda i: (0, 0)),
        ],
        [
            jax.ShapeDtypeStruct((n, h), jnp.float32),
            jax.ShapeDtypeStruct((n, 1), jnp.float32),
            jax.ShapeDtypeStruct((n, 1), jnp.float32),
            jax.ShapeDtypeStruct((8, 128), jnp.float32),
        ],
    )


def _tc_pre(x, w, a_s, a_d, blk):
    n, d = x.shape
    h = w.shape[1]
    nsteps = n // blk
    out_specs, out_shape = _dense_outs(n, h, blk)
    return pl.pallas_call(
        functools.partial(_pre_body, nsteps),
        grid=(nsteps,),
        in_specs=[
            pl.BlockSpec((blk, d), lambda i: (i, 0)),
            pl.BlockSpec((d, h), lambda i: (0, 0)),
            pl.BlockSpec((h, 1), lambda i: (0, 0)),
            pl.BlockSpec((h, 1), lambda i: (0, 0)),
        ],
        out_specs=out_specs,
        out_shape=out_shape,
        scratch_shapes=[pltpu.SMEM((2,), jnp.float32)],
    )(x, w, a_s, a_d)


def _combine(q_refs, pd_ref, b_ref):
    num = jnp.concatenate([q[0] + q[1] for q in q_refs], axis=1)
    den = (pd_ref[0] + pd_ref[1])[:, 0:1]
    return jnp.maximum(num / (den + 1e-16) + b_ref[...], 0.0)


def _mid_body(nsteps, q0, q1, q2, q3, pd_ref, b_ref,
              w_ref, as_ref, ad_ref,
              hw_ref, ss_ref, sd_ref, m_ref, mscr):
    hcur = _combine((q0, q1, q2, q3), pd_ref, b_ref)
    _pre_tail(nsteps, hcur, w_ref, as_ref, ad_ref,
              hw_ref, ss_ref, sd_ref, m_ref, mscr)


def _part_specs(blk, h):
    qspec = pl.BlockSpec((2, blk, h // 4), lambda i: (0, i, 0))
    return [qspec, qspec, qspec, qspec, qspec]


def _tc_mid(p_n, p_d, b, w, a_s, a_d, n, blk):
    h = w.shape[0]
    nsteps = n // blk
    out_specs, out_shape = _dense_outs(n, h, blk)
    return pl.pallas_call(
        functools.partial(_mid_body, nsteps),
        grid=(nsteps,),
        in_specs=_part_specs(blk, h) + [
            pl.BlockSpec((1, h), lambda i: (0, 0)),
            pl.BlockSpec((h, h), lambda i: (0, 0)),
            pl.BlockSpec((h, 1), lambda i: (0, 0)),
            pl.BlockSpec((h, 1), lambda i: (0, 0)),
        ],
        out_specs=out_specs,
        out_shape=out_shape,
        scratch_shapes=[pltpu.SMEM((2,), jnp.float32)],
    )(*p_n, p_d, b, w, a_s, a_d)


def _post_body(nsteps, g, q0, q1, q2, q3, pd_ref, b_ref,
               wih_ref, bih_ref, bhh_ref, batch_ref, wfc_ref, bfc_ref,
               pooled_ref, sn_ref):
    i = pl.program_id(0)
    h = b_ref.shape[1]
    blk = q0.shape[1]
    hcur = _combine((q0, q1, q2, q3), pd_ref, b_ref)
    gx = jnp.dot(hcur, wih_ref[...], preferred_element_type=jnp.float32) + bih_ref[...]
    bhh = bhh_ref[...]
    r = jax.nn.sigmoid(gx[:, 0:h] + bhh[:, 0:h])
    z = jax.nn.sigmoid(gx[:, h:2 * h] + bhh[:, h:2 * h])
    nn_ = jnp.tanh(gx[:, 2 * h:3 * h] + r * bhh[:, 2 * h:3 * h])
    hg = (1.0 - z) * nn_
    gids = lax.broadcasted_iota(jnp.int32, (g, blk), 0)
    onehot = (gids == batch_ref[0]).astype(jnp.float32)
    contrib = jnp.dot(onehot, hg, preferred_element_type=jnp.float32)

    @pl.when(i == 0)
    def _():
        pooled_ref[...] = contrib

    @pl.when(i > 0)
    def _():
        pooled_ref[...] += contrib

    @pl.when(i == nsteps - 1)
    def _():
        pooled = pooled_ref[...]
        sn = jnp.sum(pooled, axis=0, keepdims=True) * (1.0 / g)
        sn = jnp.dot(sn, wfc_ref[...], preferred_element_type=jnp.float32) + bfc_ref[...]
        sn_ref[...] = jnp.maximum(sn, 0.0)


def _tc_post(p_n, p_d, b, w_ih_t, b_ih, b_hh, batch_row,
             w_fc_t, b_fc, g, n, blk):
    h = b.shape[1]
    nsteps = n // blk
    batch_row = batch_row.reshape(nsteps, 1, blk)
    return pl.pallas_call(
        functools.partial(_post_body, nsteps, g),
        grid=(nsteps,),
        in_specs=_part_specs(blk, h) + [
            pl.BlockSpec((1, h), lambda i: (0, 0)),
            pl.BlockSpec((h, 3 * h), lambda i: (0, 0)),
            pl.BlockSpec((1, 3 * h), lambda i: (0, 0)),
            pl.BlockSpec((1, 3 * h), lambda i: (0, 0)),
            pl.BlockSpec((1, 1, blk), lambda i: (i, 0, 0)),
            pl.BlockSpec((h, h), lambda i: (0, 0)),
            pl.BlockSpec((1, h), lambda i: (0, 0)),
        ],
        out_specs=[
            pl.BlockSpec((g, h), lambda i: (0, 0)),
            pl.BlockSpec((1, h), lambda i: (0, 0)),
        ],
        out_shape=[
            jax.ShapeDtypeStruct((g, h), jnp.float32),
            jax.ShapeDtypeStruct((1, h), jnp.float32),
        ],
    )(*p_n, p_d, b, w_ih_t, b_ih, b_hh, batch_row, w_fc_t, b_fc)


def _make_sc_edge_pass(n, n_pad, h, nb, ept):
    np4 = n_pad // 4
    npc = np4 // 16
    nck = 32
    mesh = plsc.VectorSubcoreMesh(core_axis_name="c", subcore_axis_name="s")

    @functools.partial(
        pl.kernel,
        mesh=mesh,
        compiler_params=pltpu.CompilerParams(needs_layout_passes=False),
        out_type=jax.ShapeDtypeStruct((5, 2, np4, h), jnp.float32),
        scratch_types=[
            pltpu.VMEM((nb, _B), jnp.int32),
            pltpu.VMEM((nb, _B), jnp.int32),
            pltpu.VMEM((nb, _B), jnp.int32),
            pltpu.VMEM((n,), jnp.float32),
            pltpu.VMEM((n,), jnp.float32),
            pltpu.VMEM((16,), jnp.float32),
            pltpu.VMEM((nb, _B), jnp.float32),
            pltpu.VMEM((_B, h), jnp.float32),
            pltpu.VMEM((_B, h), jnp.float32),
            pltpu.VMEM_SHARED((np4, h), jnp.float32),
            pltpu.SemaphoreType.DMA,
        ],
    )
    def sc_kernel(hw_hbm, ssrc_hbm, sdst_hbm, m_hbm, src_hbm, dst_hbm,
                  out_n, src_v, dst_v, d4_v, ssrc_v, sdst_v, m_v,
                  exs_v, gbuf, sbuf, acc_n, sem):
        cid = lax.axis_index("c")
        sid = lax.axis_index("s")
        wid = sid * 2 + cid
        pltpu.sync_copy(src_hbm.at[wid], src_v)
        pltpu.sync_copy(dst_hbm.at[wid], dst_v)
        pltpu.sync_copy(ssrc_hbm, ssrc_v)
        pltpu.sync_copy(sdst_hbm, sdst_v)
        pltpu.sync_copy(m_hbm, m_v)

        zero16 = jnp.zeros((16,), jnp.float32)
        lanes = lax.iota(jnp.int32, 16)
        base = sid * npc

        def _zrow(bi, c_):
            for c in range(h // 16):
                sbuf[bi, pl.ds(16 * c, 16)] = zero16
            return c_

        lax.fori_loop(0, _B, _zrow, 0)
        for t in range(npc // nck):
            pltpu.sync_copy(sbuf.at[pl.ds(0, nck)],
                            acc_n.at[pl.ds(base + nck * t, nck)])
        plsc.subcore_barrier()

        def _pass(p, cp_):
            pm = jnp.minimum(p, 3)

            def _batch(j, c_):
                @pl.when(p < 4)
                def _():
                    pltpu.async_copy(hw_hbm.at[src_v.at[j]], gbuf, sem).wait()

                @pl.when(p == 0)
                def _():
                    for c in range(_B // 16):
                        si = src_v[j, pl.ds(16 * c, 16)]
                        di = dst_v[j, pl.ds(16 * c, 16)]
                        sv = plsc.load_gather(ssrc_v, [si])
                        dv = plsc.load_gather(sdst_v, [di])
                        e = sv + dv
                        e = jnp.where(e < 0.0, e * 0.2, e)
                        ex = jnp.exp(e - m_v[...])
                        pos = j * _B + 16 * c + lanes
                        ex = jnp.where(pos < ept, ex, 0.0)
                        exs_v[j, pl.ds(16 * c, 16)] = ex
                        d4_v[j, pl.ds(16 * c, 16)] = lax.shift_right_logical(
                            di, 2)

                jsplat = jnp.full((16,), j, jnp.int32)

                isd = jnp.full((16,), p, jnp.int32) >= 4

                def _row(bi, c2_):
                    bsplat = jnp.full((16,), bi, jnp.int32)
                    exb = plsc.load_gather(exs_v, [jsplat, bsplat])
                    db = plsc.load_gather(dst_v, [jsplat, bsplat])
                    quad = jnp.bitwise_and(db, 3)
                    for c in range(h // 16):
                        gv = gbuf[bi, pl.ds(32 * pm + 16 * (c % 2), 16)] * exb
                        dv_ = jnp.where(lanes == 0, exb, 0.0) if c % 2 == 0 \
                            else jnp.zeros((16,), jnp.float32)
                        val = jnp.where(isd, dv_, gv)
                        sbuf[bi, pl.ds(16 * c, 16)] = jnp.where(
                            quad == c // 2, val, 0.0)
                    return c2_

                lax.fori_loop(0, _B, _row, 0)
                pltpu.sync_copy(sbuf, acc_n.at[d4_v.at[j]], add=True)
                return c_

            lax.fori_loop(0, nb, _batch, 0)
            plsc.subcore_barrier()
            for t in range(npc // nck):
                pltpu.sync_copy(acc_n.at[pl.ds(base + nck * t, nck)],
                                out_n.at[p, cid, pl.ds(base + nck * t, nck)])

            @pl.when(p < 4)
            def _():
                def _zrow2(bi, c_):
                    for c in range(h // 16):
                        sbuf[bi, pl.ds(16 * c, 16)] = zero16
                    return c_

                lax.fori_loop(0, _B, _zrow2, 0)
                for t in range(npc // nck):
                    pltpu.sync_copy(sbuf.at[pl.ds(0, nck)],
                                    acc_n.at[pl.ds(base + nck * t, nck)])

            plsc.subcore_barrier()
            return cp_

        lax.fori_loop(0, 5, _pass, 0)

    return sc_kernel


def kernel(x, edge_index, batch, W1, a_src1, a_dst1, b1,
           W2, a_src2, a_dst2, b2, W_ih, W_hh, b_ih, b_hh, W_fc, b_fc):
    n, d = x.shape
    h = W1.shape[1]
    e = edge_index.shape[1]
    g = 128
    blk = 2000

    ept = e // _NW
    nb = (ept + _B - 1) // _B
    pad = nb * _B - ept
    src_t = jnp.pad(edge_index[0].reshape(_NW, ept), ((0, 0), (0, pad))
                    ).reshape(_NW, nb, _B)
    dst_t = jnp.pad(edge_index[1].reshape(_NW, ept), ((0, 0), (0, pad))
                    ).reshape(_NW, nb, _B)

    npc = -(-(n // 16) // _B) * _B
    n_pad = 16 * npc
    sc_pass = _make_sc_edge_pass(n, n_pad, h, nb, ept)

    def unpack(out_q):
        qs = [out_q[p].reshape(2, n_pad, h // 4) for p in range(4)]
        return qs, out_q[4].reshape(2, n_pad, h // 4)

    hw1, ss1, sd1, m1 = _tc_pre(x, W1, a_src1.reshape(h, 1),
                                a_dst1.reshape(h, 1), blk)
    pn1, pd1 = unpack(sc_pass(hw1, ss1.reshape(n), sd1.reshape(n),
                              m1.reshape(-1)[:16], src_t, dst_t))
    hw2, ss2, sd2, m2 = _tc_mid(pn1, pd1, b1.reshape(1, h),
                                W2, a_src2.reshape(h, 1),
                                a_dst2.reshape(h, 1), n, blk)
    pn2, pd2 = unpack(sc_pass(hw2, ss2.reshape(n), sd2.reshape(n),
                              m2.reshape(-1)[:16], src_t, dst_t))
    pooled, sn = _tc_post(pn2, pd2, b2.reshape(1, h),
                          W_ih.T, b_ih.reshape(1, 3 * h),
                          b_hh.reshape(1, 3 * h), batch.reshape(1, n),
                          W_fc.T, b_fc.reshape(1, h), g, n, blk)
    return jnp.concatenate([pooled, jnp.tile(sn, (g, 1))], axis=1)


# speedup vs baseline: 3.5039x; 1.1971x over previous
import functools

import jax
import jax.numpy as jnp
from jax import lax
from jax.experimental import pallas as pl
from jax.experimental.pallas import tpu as pltpu
from jax.experimental.pallas import tpu_sc as plsc

_NW = 32
_B = 64


def _pre_tail(nsteps, hcur, w_ref, as_ref, ad_ref,
              hw_ref, ss_ref, sd_ref, m_ref, mscr):
    i = pl.program_id(0)
    hw = jnp.dot(hcur, w_ref[...], preferred_element_type=jnp.float32)
    hw_ref[...] = hw
    ss = jnp.dot(hw, as_ref[...], preferred_element_type=jnp.float32)
    sd = jnp.dot(hw, ad_ref[...], preferred_element_type=jnp.float32)
    ss_ref[...] = ss
    sd_ref[...] = sd
    bs = jnp.max(ss)
    bd = jnp.max(sd)

    @pl.when(i == 0)
    def _():
        mscr[0] = bs
        mscr[1] = bd

    @pl.when(i > 0)
    def _():
        mscr[0] = jnp.maximum(mscr[0], bs)
        mscr[1] = jnp.maximum(mscr[1], bd)

    @pl.when(i == nsteps - 1)
    def _():
        m = mscr[0] + mscr[1]
        m = jnp.where(m < 0.0, 0.2 * m, m)
        m_ref[...] = jnp.full((8, 128), m, jnp.float32)


def _pre_body(nsteps, x_ref, w_ref, as_ref, ad_ref,
              hw_ref, ss_ref, sd_ref, m_ref, mscr):
    _pre_tail(nsteps, x_ref[...], w_ref, as_ref, ad_ref,
              hw_ref, ss_ref, sd_ref, m_ref, mscr)


def _dense_outs(n, h, blk):
    return (
        [
            pl.BlockSpec((blk, h), lambda i: (i, 0)),
            pl.BlockSpec((blk, 1), lambda i: (i, 0)),
            pl.BlockSpec((blk, 1), lambda i: (i, 0)),
            pl.BlockSpec((8, 128), lambda i: (0, 0)),
        ],
        [
            jax.ShapeDtypeStruct((n, h), jnp.float32),
            jax.ShapeDtypeStruct((n, 1), jnp.float32),
            jax.ShapeDtypeStruct((n, 1), jnp.float32),
            jax.ShapeDtypeStruct((8, 128), jnp.float32),
        ],
    )


def _tc_pre(x, w, a_s, a_d, blk):
    n, d = x.shape
    h = w.shape[1]
    nsteps = n // blk
    out_specs, out_shape = _dense_outs(n, h, blk)
    return pl.pallas_call(
        functools.partial(_pre_body, nsteps),
        grid=(nsteps,),
        in_specs=[
            pl.BlockSpec((blk, d), lambda i: (i, 0)),
            pl.BlockSpec((d, h), lambda i: (0, 0)),
            pl.BlockSpec((h, 1), lambda i: (0, 0)),
            pl.BlockSpec((h, 1), lambda i: (0, 0)),
        ],
        out_specs=out_specs,
        out_shape=out_shape,
        scratch_shapes=[pltpu.SMEM((2,), jnp.float32)],
    )(x, w, a_s, a_d)


def _combine(q_refs, pd_ref, b_ref):
    num = jnp.concatenate([q[0] + q[1] for q in q_refs], axis=1)
    den = (pd_ref[0] + pd_ref[1])[:, 0:1]
    return jnp.maximum(num / (den + 1e-16) + b_ref[...], 0.0)


def _mid_body(nsteps, q0, q1, q2, q3, pd_ref, b_ref,
              w_ref, as_ref, ad_ref,
              hw_ref, ss_ref, sd_ref, m_ref, mscr):
    hcur = _combine((q0, q1, q2, q3), pd_ref, b_ref)
    _pre_tail(nsteps, hcur, w_ref, as_ref, ad_ref,
              hw_ref, ss_ref, sd_ref, m_ref, mscr)


def _part_specs(blk, h):
    qspec = pl.BlockSpec((2, blk, h // 4), lambda i: (0, i, 0))
    return [qspec, qspec, qspec, qspec, qspec]


def _tc_mid(p_n, p_d, b, w, a_s, a_d, n, blk):
    h = w.shape[0]
    nsteps = n // blk
    out_specs, out_shape = _dense_outs(n, h, blk)
    return pl.pallas_call(
        functools.partial(_mid_body, nsteps),
        grid=(nsteps,),
        in_specs=_part_specs(blk, h) + [
            pl.BlockSpec((1, h), lambda i: (0, 0)),
            pl.BlockSpec((h, h), lambda i: (0, 0)),
            pl.BlockSpec((h, 1), lambda i: (0, 0)),
            pl.BlockSpec((h, 1), lambda i: (0, 0)),
        ],
        out_specs=out_specs,
        out_shape=out_shape,
        scratch_shapes=[pltpu.SMEM((2,), jnp.float32)],
    )(*p_n, p_d, b, w, a_s, a_d)


def _post_body(nsteps, g, q0, q1, q2, q3, pd_ref, b_ref,
               wih_ref, bih_ref, bhh_ref, batch_ref, wfc_ref, bfc_ref,
               pooled_ref, sn_ref):
    i = pl.program_id(0)
    h = b_ref.shape[1]
    blk = q0.shape[1]
    hcur = _combine((q0, q1, q2, q3), pd_ref, b_ref)
    gx = jnp.dot(hcur, wih_ref[...], preferred_element_type=jnp.float32) + bih_ref[...]
    bhh = bhh_ref[...]
    r = jax.nn.sigmoid(gx[:, 0:h] + bhh[:, 0:h])
    z = jax.nn.sigmoid(gx[:, h:2 * h] + bhh[:, h:2 * h])
    nn_ = jnp.tanh(gx[:, 2 * h:3 * h] + r * bhh[:, 2 * h:3 * h])
    hg = (1.0 - z) * nn_
    gids = lax.broadcasted_iota(jnp.int32, (g, blk), 0)
    onehot = (gids == batch_ref[0]).astype(jnp.float32)
    contrib = jnp.dot(onehot, hg, preferred_element_type=jnp.float32)

    @pl.when(i == 0)
    def _():
        pooled_ref[...] = contrib

    @pl.when(i > 0)
    def _():
        pooled_ref[...] += contrib

    @pl.when(i == nsteps - 1)
    def _():
        pooled = pooled_ref[...]
        sn = jnp.sum(pooled, axis=0, keepdims=True) * (1.0 / g)
        sn = jnp.dot(sn, wfc_ref[...], preferred_element_type=jnp.float32) + bfc_ref[...]
        sn_ref[...] = jnp.maximum(sn, 0.0)


def _tc_post(p_n, p_d, b, w_ih_t, b_ih, b_hh, batch_row,
             w_fc_t, b_fc, g, n, blk):
    h = b.shape[1]
    nsteps = n // blk
    batch_row = batch_row.reshape(nsteps, 1, blk)
    return pl.pallas_call(
        functools.partial(_post_body, nsteps, g),
        grid=(nsteps,),
        in_specs=_part_specs(blk, h) + [
            pl.BlockSpec((1, h), lambda i: (0, 0)),
            pl.BlockSpec((h, 3 * h), lambda i: (0, 0)),
            pl.BlockSpec((1, 3 * h), lambda i: (0, 0)),
            pl.BlockSpec((1, 3 * h), lambda i: (0, 0)),
            pl.BlockSpec((1, 1, blk), lambda i: (i, 0, 0)),
            pl.BlockSpec((h, h), lambda i: (0, 0)),
            pl.BlockSpec((1, h), lambda i: (0, 0)),
        ],
        out_specs=[
            pl.BlockSpec((g, h), lambda i: (0, 0)),
            pl.BlockSpec((1, h), lambda i: (0, 0)),
        ],
        out_shape=[
            jax.ShapeDtypeStruct((g, h), jnp.float32),
            jax.ShapeDtypeStruct((1, h), jnp.float32),
        ],
    )(*p_n, p_d, b, w_ih_t, b_ih, b_hh, batch_row, w_fc_t, b_fc)


def _make_sc_edge_pass(n, n_pad, h, nb, ept):
    np4 = n_pad // 4
    npc = np4 // 16
    nck = 32
    mesh = plsc.VectorSubcoreMesh(core_axis_name="c", subcore_axis_name="s")

    @functools.partial(
        pl.kernel,
        mesh=mesh,
        compiler_params=pltpu.CompilerParams(needs_layout_passes=False),
        out_type=jax.ShapeDtypeStruct((5, 2, np4, h), jnp.float32),
        scratch_types=[
            pltpu.VMEM((nb, _B), jnp.int32),
            pltpu.VMEM((nb, _B), jnp.int32),
            pltpu.VMEM((2, _B), jnp.int32),
            pltpu.VMEM((n,), jnp.float32),
            pltpu.VMEM((n,), jnp.float32),
            pltpu.VMEM((16,), jnp.float32),
            pltpu.VMEM((2, _B), jnp.float32),
            pltpu.VMEM((_B, h), jnp.float32),
            pltpu.VMEM((_B, h), jnp.float32),
            pltpu.VMEM((_B, h), jnp.float32),
            pltpu.VMEM((_B, h), jnp.float32),
            pltpu.VMEM_SHARED((np4, h), jnp.float32),
            pltpu.SemaphoreType.DMA,
            pltpu.SemaphoreType.DMA,
            pltpu.SemaphoreType.DMA,
            pltpu.SemaphoreType.DMA,
        ],
    )
    def sc_kernel(hw_hbm, ssrc_hbm, sdst_hbm, m_hbm, src_hbm, dst_hbm,
                  out_n, src_v, dst_v, d4r_v, ssrc_v, sdst_v, m_v,
                  exr_v, gbuf0, gbuf1, sbuf0, sbuf1, acc_n,
                  gsem0, gsem1, ssem0, ssem1):
        cid = lax.axis_index("c")
        sid = lax.axis_index("s")
        wid = sid * 2 + cid
        pltpu.sync_copy(src_hbm.at[wid], src_v)
        pltpu.sync_copy(dst_hbm.at[wid], dst_v)
        pltpu.sync_copy(ssrc_hbm, ssrc_v)
        pltpu.sync_copy(sdst_hbm, sdst_v)
        pltpu.sync_copy(m_hbm, m_v)

        zero16 = jnp.zeros((16,), jnp.float32)
        lanes = lax.iota(jnp.int32, 16)
        base = sid * npc

        def _zrow(bi, c_):
            for c in range(h // 16):
                sbuf0[bi, pl.ds(16 * c, 16)] = zero16
            return c_

        lax.fori_loop(0, _B, _zrow, 0)
        for t in range(npc // nck):
            pltpu.sync_copy(sbuf0.at[pl.ds(0, nck)],
                            acc_n.at[pl.ds(base + nck * t, nck)])
        plsc.subcore_barrier()
        gbufs = (gbuf0, gbuf1)
        sbufs = (sbuf0, sbuf1)
        gsems = (gsem0, gsem1)
        ssems = (ssem0, ssem1)

        def _pass(p, cp_):
            pm = jnp.minimum(p, 3)
            isd = jnp.full((16,), p, jnp.int32) >= 4

            @pl.when(p < 4)
            def _():
                pltpu.async_copy(hw_hbm.at[src_v.at[0]], gbuf0, gsem0)

            def _one(j, u):
                gbuf = gbufs[u]
                sbuf = sbufs[u]

                @pl.when(jnp.logical_and(p < 4, j + 1 < nb))
                def _():
                    pltpu.async_copy(hw_hbm.at[src_v.at[j + 1]],
                                     gbufs[1 - u], gsems[1 - u])

                @pl.when(p < 4)
                def _():
                    pltpu.make_async_copy(hw_hbm.at[pl.ds(0, _B)], gbuf,
                                          gsems[u]).wait()

                @pl.when(j >= 2)
                def _():
                    pltpu.make_async_copy(hw_hbm.at[pl.ds(0, _B)], sbuf,
                                          ssems[u]).wait()

                for c in range(_B // 16):
                    si = src_v[j, pl.ds(16 * c, 16)]
                    di = dst_v[j, pl.ds(16 * c, 16)]
                    sv = plsc.load_gather(ssrc_v, [si])
                    dv = plsc.load_gather(sdst_v, [di])
                    e = sv + dv
                    e = jnp.where(e < 0.0, e * 0.2, e)
                    ex = jnp.exp(e - m_v[...])
                    pos = j * _B + 16 * c + lanes
                    ex = jnp.where(pos < ept, ex, 0.0)
                    exr_v[u, pl.ds(16 * c, 16)] = ex
                    d4r_v[u, pl.ds(16 * c, 16)] = lax.shift_right_logical(
                        di, 2)

                usplat = jnp.full((16,), u, jnp.int32)
                jsplat = jnp.full((16,), j, jnp.int32)

                def _row(bi, c2_):
                    bsplat = jnp.full((16,), bi, jnp.int32)
                    exb = plsc.load_gather(exr_v, [usplat, bsplat])
                    db = plsc.load_gather(dst_v, [jsplat, bsplat])
                    quad = jnp.bitwise_and(db, 3)
                    for c in range(h // 16):
                        gv = gbuf[bi, pl.ds(32 * pm + 16 * (c % 2), 16)] * exb
                        dv_ = jnp.where(lanes == 0, exb, 0.0) if c % 2 == 0 \
                            else jnp.zeros((16,), jnp.float32)
                        val = jnp.where(isd, dv_, gv)
                        sbuf[bi, pl.ds(16 * c, 16)] = jnp.where(
                            quad == c // 2, val, 0.0)
                    return c2_

                lax.fori_loop(0, _B, _row, 0)
                pltpu.async_copy(sbuf, acc_n.at[d4r_v.at[u]], ssems[u],
                                 add=True)

            def _batch(jj, c_):
                _one(jj * 2, 0)
                _one(jj * 2 + 1, 1)
                return c_

            lax.fori_loop(0, nb // 2, _batch, 0)
            for u in range(2):
                pltpu.make_async_copy(hw_hbm.at[pl.ds(0, _B)], sbufs[u],
                                      ssems[u]).wait()
            plsc.subcore_barrier()
            for t in range(npc // nck):
                pltpu.sync_copy(acc_n.at[pl.ds(base + nck * t, nck)],
                                out_n.at[p, cid, pl.ds(base + nck * t, nck)])

            @pl.when(p < 4)
            def _():
                def _zrow2(bi, c_):
                    for c in range(h // 16):
                        sbuf0[bi, pl.ds(16 * c, 16)] = zero16
                    return c_

                lax.fori_loop(0, _B, _zrow2, 0)
                for t in range(npc // nck):
                    pltpu.sync_copy(sbuf0.at[pl.ds(0, nck)],
                                    acc_n.at[pl.ds(base + nck * t, nck)])

            plsc.subcore_barrier()
            return cp_

        lax.fori_loop(0, 5, _pass, 0)

    return sc_kernel


def kernel(x, edge_index, batch, W1, a_src1, a_dst1, b1,
           W2, a_src2, a_dst2, b2, W_ih, W_hh, b_ih, b_hh, W_fc, b_fc):
    n, d = x.shape
    h = W1.shape[1]
    e = edge_index.shape[1]
    g = 128
    blk = 2000

    ept = e // _NW
    nb = (ept + _B - 1) // _B
    nb += nb % 2
    pad = nb * _B - ept
    src_t = jnp.pad(edge_index[0].reshape(_NW, ept), ((0, 0), (0, pad))
                    ).reshape(_NW, nb, _B)
    dst_t = jnp.pad(edge_index[1].reshape(_NW, ept), ((0, 0), (0, pad))
                    ).reshape(_NW, nb, _B)

    npc = -(-(n // 16) // _B) * _B
    n_pad = 16 * npc
    sc_pass = _make_sc_edge_pass(n, n_pad, h, nb, ept)

    def unpack(out_q):
        qs = [out_q[p].reshape(2, n_pad, h // 4) for p in range(4)]
        return qs, out_q[4].reshape(2, n_pad, h // 4)

    hw1, ss1, sd1, m1 = _tc_pre(x, W1, a_src1.reshape(h, 1),
                                a_dst1.reshape(h, 1), blk)
    pn1, pd1 = unpack(sc_pass(hw1, ss1.reshape(n), sd1.reshape(n),
                              m1.reshape(-1)[:16], src_t, dst_t))
    hw2, ss2, sd2, m2 = _tc_mid(pn1, pd1, b1.reshape(1, h),
                                W2, a_src2.reshape(h, 1),
                                a_dst2.reshape(h, 1), n, blk)
    pn2, pd2 = unpack(sc_pass(hw2, ss2.reshape(n), sd2.reshape(n),
                              m2.reshape(-1)[:16], src_t, dst_t))
    pooled, sn = _tc_post(pn2, pd2, b2.reshape(1, h),
                          W_ih.T, b_ih.reshape(1, 3 * h),
                          b_hh.reshape(1, 3 * h), batch.reshape(1, n),
                          W_fc.T, b_fc.reshape(1, h), g, n, blk)
    return jnp.concatenate([pooled, jnp.tile(sn, (g, 1))], axis=1)


# speedup vs baseline: 6.0291x; 1.7207x over previous
import functools

import jax
import jax.numpy as jnp
from jax import lax
from jax.experimental import pallas as pl
from jax.experimental.pallas import tpu as pltpu
from jax.experimental.pallas import tpu_sc as plsc

_NW = 32
_B = 64


def _pre_tail(nsteps, hcur, w_ref, as_ref, ad_ref,
              hw_ref, ss_ref, sd_ref, m_ref, mscr):
    i = pl.program_id(0)
    hw = jnp.dot(hcur, w_ref[...], preferred_element_type=jnp.float32)
    hw_ref[...] = hw
    ss = jnp.dot(hw, as_ref[...], preferred_element_type=jnp.float32)
    sd = jnp.dot(hw, ad_ref[...], preferred_element_type=jnp.float32)
    ss_ref[...] = ss
    sd_ref[...] = sd
    bs = jnp.max(ss)
    bd = jnp.max(sd)

    @pl.when(i == 0)
    def _():
        mscr[0] = bs
        mscr[1] = bd

    @pl.when(i > 0)
    def _():
        mscr[0] = jnp.maximum(mscr[0], bs)
        mscr[1] = jnp.maximum(mscr[1], bd)

    @pl.when(i == nsteps - 1)
    def _():
        m = mscr[0] + mscr[1]
        m = jnp.where(m < 0.0, 0.2 * m, m)
        m_ref[...] = jnp.full((8, 128), m, jnp.float32)


def _pre_body(nsteps, x_ref, w_ref, as_ref, ad_ref,
              hw_ref, ss_ref, sd_ref, m_ref, mscr):
    _pre_tail(nsteps, x_ref[...], w_ref, as_ref, ad_ref,
              hw_ref, ss_ref, sd_ref, m_ref, mscr)


def _dense_outs(n, h, blk):
    return (
        [
            pl.BlockSpec((blk, h), lambda i: (i, 0)),
            pl.BlockSpec((blk, 1), lambda i: (i, 0)),
            pl.BlockSpec((blk, 1), lambda i: (i, 0)),
            pl.BlockSpec((8, 128), lambda i: (0, 0)),
        ],
        [
            jax.ShapeDtypeStruct((n, h), jnp.float32),
            jax.ShapeDtypeStruct((n, 1), jnp.float32),
            jax.ShapeDtypeStruct((n, 1), jnp.float32),
            jax.ShapeDtypeStruct((8, 128), jnp.float32),
        ],
    )


def _tc_pre(x, w, a_s, a_d, blk):
    n, d = x.shape
    h = w.shape[1]
    nsteps = n // blk
    out_specs, out_shape = _dense_outs(n, h, blk)
    return pl.pallas_call(
        functools.partial(_pre_body, nsteps),
        grid=(nsteps,),
        in_specs=[
            pl.BlockSpec((blk, d), lambda i: (i, 0)),
            pl.BlockSpec((d, h), lambda i: (0, 0)),
            pl.BlockSpec((h, 1), lambda i: (0, 0)),
            pl.BlockSpec((h, 1), lambda i: (0, 0)),
        ],
        out_specs=out_specs,
        out_shape=out_shape,
        scratch_shapes=[pltpu.SMEM((2,), jnp.float32)],
    )(x, w, a_s, a_d)


def _combine(q_refs, pd_ref, b_ref):
    num = jnp.concatenate([q[0] + q[1] for q in q_refs], axis=1)
    den = (pd_ref[0] + pd_ref[1])[:, 0:1]
    return jnp.maximum(num / (den + 1e-16) + b_ref[...], 0.0)


def _mid_body(nsteps, q0, q1, pd_ref, b_ref,
              w_ref, as_ref, ad_ref,
              hw_ref, ss_ref, sd_ref, m_ref, mscr):
    hcur = _combine((q0, q1), pd_ref, b_ref)
    _pre_tail(nsteps, hcur, w_ref, as_ref, ad_ref,
              hw_ref, ss_ref, sd_ref, m_ref, mscr)


def _part_specs(blk, h):
    qspec = pl.BlockSpec((2, blk, h // 2), lambda i: (0, i, 0))
    return [qspec, qspec, qspec]


def _tc_mid(p_n, p_d, b, w, a_s, a_d, n, blk):
    h = w.shape[0]
    nsteps = n // blk
    out_specs, out_shape = _dense_outs(n, h, blk)
    return pl.pallas_call(
        functools.partial(_mid_body, nsteps),
        grid=(nsteps,),
        in_specs=_part_specs(blk, h) + [
            pl.BlockSpec((1, h), lambda i: (0, 0)),
            pl.BlockSpec((h, h), lambda i: (0, 0)),
            pl.BlockSpec((h, 1), lambda i: (0, 0)),
            pl.BlockSpec((h, 1), lambda i: (0, 0)),
        ],
        out_specs=out_specs,
        out_shape=out_shape,
        scratch_shapes=[pltpu.SMEM((2,), jnp.float32)],
    )(*p_n, p_d, b, w, a_s, a_d)


def _post_body(nsteps, g, q0, q1, pd_ref, b_ref,
               wih_ref, bih_ref, bhh_ref, batch_ref, wfc_ref, bfc_ref,
               pooled_ref, sn_ref):
    i = pl.program_id(0)
    h = b_ref.shape[1]
    blk = q0.shape[1]
    hcur = _combine((q0, q1), pd_ref, b_ref)
    gx = jnp.dot(hcur, wih_ref[...], preferred_element_type=jnp.float32) + bih_ref[...]
    bhh = bhh_ref[...]
    r = jax.nn.sigmoid(gx[:, 0:h] + bhh[:, 0:h])
    z = jax.nn.sigmoid(gx[:, h:2 * h] + bhh[:, h:2 * h])
    nn_ = jnp.tanh(gx[:, 2 * h:3 * h] + r * bhh[:, 2 * h:3 * h])
    hg = (1.0 - z) * nn_
    gids = lax.broadcasted_iota(jnp.int32, (g, blk), 0)
    onehot = (gids == batch_ref[0]).astype(jnp.float32)
    contrib = jnp.dot(onehot, hg, preferred_element_type=jnp.float32)

    @pl.when(i == 0)
    def _():
        pooled_ref[...] = contrib

    @pl.when(i > 0)
    def _():
        pooled_ref[...] += contrib

    @pl.when(i == nsteps - 1)
    def _():
        pooled = pooled_ref[...]
        sn = jnp.sum(pooled, axis=0, keepdims=True) * (1.0 / g)
        sn = jnp.dot(sn, wfc_ref[...], preferred_element_type=jnp.float32) + bfc_ref[...]
        sn_ref[...] = jnp.maximum(sn, 0.0)


def _tc_post(p_n, p_d, b, w_ih_t, b_ih, b_hh, batch_row,
             w_fc_t, b_fc, g, n, blk):
    h = b.shape[1]
    nsteps = n // blk
    batch_row = batch_row.reshape(nsteps, 1, blk)
    return pl.pallas_call(
        functools.partial(_post_body, nsteps, g),
        grid=(nsteps,),
        in_specs=_part_specs(blk, h) + [
            pl.BlockSpec((1, h), lambda i: (0, 0)),
            pl.BlockSpec((h, 3 * h), lambda i: (0, 0)),
            pl.BlockSpec((1, 3 * h), lambda i: (0, 0)),
            pl.BlockSpec((1, 3 * h), lambda i: (0, 0)),
            pl.BlockSpec((1, 1, blk), lambda i: (i, 0, 0)),
            pl.BlockSpec((h, h), lambda i: (0, 0)),
            pl.BlockSpec((1, h), lambda i: (0, 0)),
        ],
        out_specs=[
            pl.BlockSpec((g, h), lambda i: (0, 0)),
            pl.BlockSpec((1, h), lambda i: (0, 0)),
        ],
        out_shape=[
            jax.ShapeDtypeStruct((g, h), jnp.float32),
            jax.ShapeDtypeStruct((1, h), jnp.float32),
        ],
    )(*p_n, p_d, b, w_ih_t, b_ih, b_hh, batch_row, w_fc_t, b_fc)


def _make_sc_edge_pass(n, n_pad, h, nb, ept):
    np2 = n_pad // 2
    npc = np2 // 16
    nck = 64
    hq = h // 2
    mesh = plsc.VectorSubcoreMesh(core_axis_name="c", subcore_axis_name="s")

    @functools.partial(
        pl.kernel,
        mesh=mesh,
        compiler_params=pltpu.CompilerParams(needs_layout_passes=False),
        out_type=jax.ShapeDtypeStruct((3, 2, np2, h), jnp.float32),
        scratch_types=[
            pltpu.VMEM((2, _B), jnp.int32),
            pltpu.VMEM((2, _B), jnp.int32),
            pltpu.VMEM((2, _B), jnp.int32),
            pltpu.VMEM((2, _B), jnp.float32),
            pltpu.VMEM((n,), jnp.float32),
            pltpu.VMEM((n,), jnp.float32),
            pltpu.VMEM((16,), jnp.float32),
            pltpu.VMEM((_B, h), jnp.float32),
            pltpu.VMEM((_B, h), jnp.float32),
            pltpu.VMEM((_B, h), jnp.float32),
            pltpu.VMEM((_B, h), jnp.float32),
            pltpu.VMEM_SHARED((np2, h), jnp.float32),
            pltpu.SemaphoreType.DMA,
            pltpu.SemaphoreType.DMA,
            pltpu.SemaphoreType.DMA,
            pltpu.SemaphoreType.DMA,
            pltpu.SemaphoreType.DMA,
        ],
    )
    def sc_kernel(hw_hbm, ssrc_hbm, sdst_hbm, m_hbm, src_hbm, dst_hbm,
                  out_n, srcr_v, dstr_v, d2r_v, exr_v, ssrc_v, sdst_v, m_v,
                  gbuf0, gbuf1, sbuf0, sbuf1, acc_n,
                  gsem0, gsem1, ssem0, ssem1, isem):
        cid = lax.axis_index("c")
        sid = lax.axis_index("s")
        wid = sid * 2 + cid
        pltpu.sync_copy(ssrc_hbm, ssrc_v)
        pltpu.sync_copy(sdst_hbm, sdst_v)
        pltpu.sync_copy(m_hbm, m_v)

        zero16 = jnp.zeros((16,), jnp.float32)
        lanes = lax.iota(jnp.int32, 16)
        base = sid * npc
        gbufs = (gbuf0, gbuf1)
        sbufs = (sbuf0, sbuf1)
        gsems = (gsem0, gsem1)
        ssems = (ssem0, ssem1)

        def _zrow(bi, c_):
            for c in range(h // 16):
                sbuf0[bi, pl.ds(16 * c, 16)] = zero16
            return c_

        lax.fori_loop(0, _B, _zrow, 0)
        for t in range(npc // nck):
            pltpu.sync_copy(sbuf0.at[pl.ds(0, nck)],
                            acc_n.at[pl.ds(base + nck * t, nck)])
        plsc.subcore_barrier()

        def _stage_idx(j, u, sem):
            pltpu.async_copy(src_hbm.at[wid, j], srcr_v.at[u], sem)
            pltpu.async_copy(dst_hbm.at[wid, j], dstr_v.at[u], sem)

        def _wait_idx():
            pltpu.make_async_copy(src_hbm.at[wid, 0], srcr_v.at[0],
                                  isem).wait()
            pltpu.make_async_copy(dst_hbm.at[wid, 0], dstr_v.at[0],
                                  isem).wait()

        def _pass(p, cp_):
            pm = jnp.minimum(p, 1)
            isd = jnp.full((16,), p, jnp.int32) >= 2

            pltpu.sync_copy(src_hbm.at[wid, 0], srcr_v.at[0])
            pltpu.sync_copy(dst_hbm.at[wid, 0], dstr_v.at[0])
            _stage_idx(1, 1, isem)

            @pl.when(p < 2)
            def _():
                pltpu.async_copy(hw_hbm.at[srcr_v.at[0]], gbuf0, gsem0)

            def _one(j, u):
                gbuf = gbufs[u]
                sbuf = sbufs[u]

                @pl.when(j + 1 < nb)
                def _():
                    _wait_idx()

                    @pl.when(p < 2)
                    def _():
                        pltpu.async_copy(hw_hbm.at[srcr_v.at[1 - u]],
                                         gbufs[1 - u], gsems[1 - u])

                @pl.when(p < 2)
                def _():
                    pltpu.make_async_copy(hw_hbm.at[pl.ds(0, _B)], gbuf,
                                          gsems[u]).wait()

                @pl.when(j >= 2)
                def _():
                    pltpu.make_async_copy(hw_hbm.at[pl.ds(0, _B)], sbuf,
                                          ssems[u]).wait()

                for c in range(_B // 16):
                    si = srcr_v[u, pl.ds(16 * c, 16)]
                    di = dstr_v[u, pl.ds(16 * c, 16)]
                    sv = plsc.load_gather(ssrc_v, [si])
                    dv = plsc.load_gather(sdst_v, [di])
                    e = sv + dv
                    e = jnp.where(e < 0.0, e * 0.2, e)
                    ex = jnp.exp(e - m_v[...])
                    pos = j * _B + 16 * c + lanes
                    ex = jnp.where(pos < ept, ex, 0.0)
                    par = jnp.bitwise_and(di, 1)
                    exr_v[u, pl.ds(16 * c, 16)] = jnp.where(
                        par == 1, -ex, ex)
                    d2r_v[u, pl.ds(16 * c, 16)] = lax.shift_right_logical(
                        di, 1)

                @pl.when(j + 2 < nb)
                def _():
                    _stage_idx(j + 2, u, isem)

                usplat = jnp.full((16,), u, jnp.int32)

                def _row(bi, c2_):
                    bsplat = jnp.full((16,), bi, jnp.int32)
                    exs = plsc.load_gather(exr_v, [usplat, bsplat])
                    odd = exs < 0.0
                    exb = jnp.abs(exs)
                    for c in range(h // 16):
                        gv = gbuf[bi, pl.ds(hq * pm + 16 * (c % 4), 16)] * exb
                        dv_ = jnp.where(lanes == 0, exb, 0.0) if c % 4 == 0 \
                            else jnp.zeros((16,), jnp.float32)
                        val = jnp.where(isd, dv_, gv)
                        sel = odd if c >= 4 else jnp.logical_not(odd)
                        sbuf[bi, pl.ds(16 * c, 16)] = jnp.where(
                            sel, val, 0.0)
                    return c2_

                lax.fori_loop(0, _B, _row, 0)
                pltpu.async_copy(sbuf, acc_n.at[d2r_v.at[u]], ssems[u],
                                 add=True)

            def _batch(jj, c_):
                _one(jj * 2, 0)
                _one(jj * 2 + 1, 1)
                return c_

            lax.fori_loop(0, nb // 2, _batch, 0)
            for u in range(2):
                pltpu.make_async_copy(hw_hbm.at[pl.ds(0, _B)], sbufs[u],
                                      ssems[u]).wait()
            plsc.subcore_barrier()
            for t in range(npc // nck):
                pltpu.sync_copy(acc_n.at[pl.ds(base + nck * t, nck)],
                                out_n.at[p, cid, pl.ds(base + nck * t, nck)])

            @pl.when(p < 2)
            def _():
                def _zrow2(bi, c_):
                    for c in range(h // 16):
                        sbuf0[bi, pl.ds(16 * c, 16)] = zero16
                    return c_

                lax.fori_loop(0, _B, _zrow2, 0)
                for t in range(npc // nck):
                    pltpu.sync_copy(sbuf0.at[pl.ds(0, nck)],
                                    acc_n.at[pl.ds(base + nck * t, nck)])

            plsc.subcore_barrier()
            return cp_

        lax.fori_loop(0, 3, _pass, 0)

    return sc_kernel


def kernel(x, edge_index, batch, W1, a_src1, a_dst1, b1,
           W2, a_src2, a_dst2, b2, W_ih, W_hh, b_ih, b_hh, W_fc, b_fc):
    n, d = x.shape
    h = W1.shape[1]
    e = edge_index.shape[1]
    g = 128
    blk = 2000

    ept = e // _NW
    nb = (ept + _B - 1) // _B
    nb += nb % 2
    pad = nb * _B - ept
    src_t = jnp.pad(edge_index[0].reshape(_NW, ept), ((0, 0), (0, pad))
                    ).reshape(_NW, nb, _B)
    dst_t = jnp.pad(edge_index[1].reshape(_NW, ept), ((0, 0), (0, pad))
                    ).reshape(_NW, nb, _B)

    npc = -(-(n // 16) // _B) * _B
    n_pad = 16 * npc
    sc_pass = _make_sc_edge_pass(n, n_pad, h, nb, ept)

    def unpack(out_q):
        qs = [out_q[p].reshape(2, n_pad, h // 2) for p in range(2)]
        return qs, out_q[2].reshape(2, n_pad, h // 2)

    hw1, ss1, sd1, m1 = _tc_pre(x, W1, a_src1.reshape(h, 1),
                                a_dst1.reshape(h, 1), blk)
    pn1, pd1 = unpack(sc_pass(hw1, ss1.reshape(n), sd1.reshape(n),
                              m1.reshape(-1)[:16], src_t, dst_t))
    hw2, ss2, sd2, m2 = _tc_mid(pn1, pd1, b1.reshape(1, h),
                                W2, a_src2.reshape(h, 1),
                                a_dst2.reshape(h, 1), n, blk)
    pn2, pd2 = unpack(sc_pass(hw2, ss2.reshape(n), sd2.reshape(n),
                              m2.reshape(-1)[:16], src_t, dst_t))
    pooled, sn = _tc_post(pn2, pd2, b2.reshape(1, h),
                          W_ih.T, b_ih.reshape(1, 3 * h),
                          b_hh.reshape(1, 3 * h), batch.reshape(1, n),
                          W_fc.T, b_fc.reshape(1, h), g, n, blk)
    return jnp.concatenate([pooled, jnp.tile(sn, (g, 1))], axis=1)
